# baseline (speedup 1.0000x reference)
import functools

import jax
import jax.numpy as jnp
from jax import lax
from jax.experimental import pallas as pl
from jax.experimental.pallas import tpu as pltpu
from jax.experimental.pallas import tpu_sc as plsc

B = 16384
NC, NS = 2, 16
NW = NC * NS
BPW = B // NW
CHUNK = 128
NCHUNK = BPW // CHUNK


def _sc_gather_body(uids, iids, gut, git, mut, mit,
                    gu_o, gi_o, mu_o, mi_o,
                    uix, iix, guv, giv, muv, miv, sem):
    wid = lax.axis_index("s") * NC + lax.axis_index("c")
    base = wid * BPW
    for c in range(NCHUNK):
        pltpu.sync_copy(uids.at[pl.ds(base + c * CHUNK, CHUNK)], uix.at[c])
        pltpu.sync_copy(iids.at[pl.ds(base + c * CHUNK, CHUNK)], iix.at[c])
    copies = []
    for c in range(NCHUNK):
        sl = pl.ds(c * CHUNK, CHUNK)
        copies.append(pltpu.async_copy(gut.at[uix.at[c]], guv.at[sl], sem))
        copies.append(pltpu.async_copy(git.at[iix.at[c]], giv.at[sl], sem))
        copies.append(pltpu.async_copy(mut.at[uix.at[c]], muv.at[sl], sem))
        copies.append(pltpu.async_copy(mit.at[iix.at[c]], miv.at[sl], sem))
    for cp in copies:
        cp.wait()
    pltpu.sync_copy(guv, gu_o.at[pl.ds(base, BPW)])
    pltpu.sync_copy(giv, gi_o.at[pl.ds(base, BPW)])
    pltpu.sync_copy(muv, mu_o.at[pl.ds(base, BPW)])
    pltpu.sync_copy(miv, mi_o.at[pl.ds(base, BPW)])


_sc_gather = functools.partial(
    pl.kernel,
    out_type=(
        jax.ShapeDtypeStruct((B, 32), jnp.float32),
        jax.ShapeDtypeStruct((B, 32), jnp.float32),
        jax.ShapeDtypeStruct((B, 16), jnp.float32),
        jax.ShapeDtypeStruct((B, 16), jnp.float32),
    ),
    mesh=plsc.VectorSubcoreMesh(core_axis_name="c", subcore_axis_name="s"),
    scratch_types=[
        pltpu.VMEM((NCHUNK, CHUNK), jnp.int32),
        pltpu.VMEM((NCHUNK, CHUNK), jnp.int32),
        pltpu.VMEM((BPW, 32), jnp.float32),
        pltpu.VMEM((BPW, 32), jnp.float32),
        pltpu.VMEM((BPW, 16), jnp.float32),
        pltpu.VMEM((BPW, 16), jnp.float32),
        pltpu.SemaphoreType.DMA,
    ],
    compiler_params=pltpu.CompilerParams(use_tc_tiling_on_sc=False),
)(_sc_gather_body)


def _dense_body(gu, gi, mu, mi, w1a, w1b, b1, w2, b2, w3, b3, wpg, wph, bp, out):
    h = jnp.dot(mu[...], w1a[...], preferred_element_type=jnp.float32)
    h += jnp.dot(mi[...], w1b[...], preferred_element_type=jnp.float32)
    h = jnp.maximum(h + b1[...], 0.0)
    h = jnp.maximum(jnp.dot(h, w2[...], preferred_element_type=jnp.float32) + b2[...], 0.0)
    h = jnp.maximum(jnp.dot(h, w3[...], preferred_element_type=jnp.float32) + b3[...], 0.0)
    g = gu[...] * gi[...]
    logit = jnp.sum(g * wpg[...], axis=1, keepdims=True)
    logit += jnp.sum(h * wph[...], axis=1, keepdims=True)
    logit += bp[...]
    out[...] = 1.0 / (1.0 + jnp.exp(-logit))


def kernel(user_ids, item_ids, gmf_user_table, gmf_item_table,
           mlp_user_table, mlp_item_table, W1, b1, W2, b2, W3, b3, Wp, bp):
    gu, gi, mu, mi = _sc_gather(user_ids, item_ids, gmf_user_table,
                                gmf_item_table, mlp_user_table, mlp_item_table)
    w1a, w1b = W1[:16, :], W1[16:, :]
    wpg = Wp[:32, 0].reshape(1, 32)
    wph = Wp[32:, 0].reshape(1, 8)
    out = pl.pallas_call(
        _dense_body,
        out_shape=jax.ShapeDtypeStruct((B, 1), jnp.float32),
    )(gu, gi, mu, mi, w1a, w1b, b1.reshape(1, 32), W2, b2.reshape(1, 16),
      W3, b3.reshape(1, 8), wpg, wph, bp.reshape(1, 1))
    return out
